# baseline (speedup 1.0000x reference)
import jax
import jax.numpy as jnp
from jax.experimental import pallas as pl
from jax.experimental.pallas import tpu as pltpu


def _router_body(x_ref, wr_ref, eb_ref, gates_ref):
    x = x_ref[...]
    wr = wr_ref[...]
    logits = jax.lax.dot_general(
        x, wr, (((1,), (1,)), ((), ())), preferred_element_type=jnp.float32)
    logits = logits + eb_ref[...]
    ne = logits.shape[1]
    m1 = jnp.max(logits, axis=1, keepdims=True)
    iota = jax.lax.broadcasted_iota(jnp.int32, logits.shape, 1)
    am = jnp.min(jnp.where(logits == m1, iota, ne), axis=1, keepdims=True)
    m2 = jnp.max(jnp.where(iota == am, -jnp.inf, logits), axis=1, keepdims=True)
    selm = logits >= m2
    ex = jnp.where(selm, jnp.exp(logits - m1), 0.0)
    gates_ref[...] = ex / jnp.sum(ex, axis=1, keepdims=True)


def _moe_body(x_ref, w1_ref, b1_ref, w2_ref, b2_ref, g_ref, out_ref, acc_ref,
              *, n_experts, n_chunks):
    e = pl.program_id(0)
    c = pl.program_id(1)

    @pl.when(jnp.logical_and(e == 0, c == 0))
    def _():
        acc_ref[...] = jnp.zeros_like(acc_ref)

    x = x_ref[...]
    w1 = w1_ref[0]
    h = jax.lax.dot_general(
        x, w1, (((1,), (1,)), ((), ())), preferred_element_type=jnp.float32)
    h = jnp.maximum(h + b1_ref[0, 0], 0.0)
    hb = h.astype(jnp.bfloat16)
    w2 = w2_ref[0]
    y = jax.lax.dot_general(
        hb, w2, (((1,), (1,)), ((), ())), preferred_element_type=jnp.float32)
    b2_scale = jnp.where(c == 0, 1.0, 0.0).astype(jnp.float32)
    y = y + b2_scale * b2_ref[0]
    g = g_ref[0].reshape(-1, 1)
    acc_ref[...] += g * y

    @pl.when(jnp.logical_and(e == n_experts - 1, c == n_chunks - 1))
    def _():
        out_ref[...] = acc_ref[...]


def kernel(x, sW1, sb1, sW2, sb2, rW1, rb1, rW2, rb2, Wr, expert_bias):
    B, T, C = x.shape
    E, DFF, _ = rW1.shape
    NE = E + sW1.shape[0]
    CCHUNK = 768
    NC = DFF // CCHUNK
    RB = 256

    x_flat = x.reshape(T, C)

    gates = pl.pallas_call(
        _router_body,
        grid=(T // RB,),
        in_specs=[
            pl.BlockSpec((RB, C), lambda i: (i, 0)),
            pl.BlockSpec((E, C), lambda i: (0, 0)),
            pl.BlockSpec((1, E), lambda i: (0, 0)),
        ],
        out_specs=pl.BlockSpec((RB, E), lambda i: (i, 0)),
        out_shape=jax.ShapeDtypeStruct((T, E), jnp.float32),
    )(x_flat, Wr, expert_bias.reshape(1, E))

    W1_all = jnp.concatenate([rW1, sW1]).astype(jnp.bfloat16)
    b1_all = jnp.concatenate([rb1, sb1]).reshape(NE, NC, 1, CCHUNK)
    W2_all = jnp.concatenate([rW2, sW2]).astype(jnp.bfloat16)
    b2_all = jnp.concatenate([rb2, sb2]).reshape(NE, 1, C)
    g_all = jnp.concatenate(
        [gates.T, jnp.ones((sW1.shape[0], T), jnp.float32)], axis=0
    ).reshape(NE, 1, T)
    x_bf = x_flat.astype(jnp.bfloat16)

    import functools
    body = functools.partial(_moe_body, n_experts=NE, n_chunks=NC)
    out = pl.pallas_call(
        body,
        grid=(NE, NC),
        in_specs=[
            pl.BlockSpec((T, C), lambda e, c: (0, 0)),
            pl.BlockSpec((1, CCHUNK, C), lambda e, c: (e, c, 0)),
            pl.BlockSpec((1, 1, 1, CCHUNK), lambda e, c: (e, c, 0, 0)),
            pl.BlockSpec((1, C, CCHUNK), lambda e, c: (e, 0, c)),
            pl.BlockSpec((1, 1, C), lambda e, c: (e, 0, 0)),
            pl.BlockSpec((1, 1, T), lambda e, c: (e, 0, 0)),
        ],
        out_specs=pl.BlockSpec((T, C), lambda e, c: (0, 0)),
        out_shape=jax.ShapeDtypeStruct((T, C), jnp.float32),
        scratch_shapes=[pltpu.VMEM((T, C), jnp.float32)],
    )(x_bf, W1_all, b1_all, W2_all, b2_all, g_all)

    return out.reshape(B, T, C)
